# baseline (speedup 1.0000x reference)
import functools

import jax
import jax.numpy as jnp
from jax import lax
from jax.experimental import pallas as pl
from jax.experimental.pallas import tpu as pltpu
from jax.experimental.pallas import tpu_sc as plsc

_EPS = 1e-5
_NS = 16
_NW = 32


def _fps_body(px_ref, py_ref, pz_ref, idx_ref, sx_ref, sy_ref, sz_ref, d_ref):
    B, R, C = px_ref.shape
    M = idx_ref.shape[0]
    lin = (lax.broadcasted_iota(jnp.int32, (R, C), 0) * C
           + lax.broadcasted_iota(jnp.int32, (R, C), 1))
    BIG = jnp.int32(1 << 30)

    def pick(ref_b, sel):
        return jnp.sum(jnp.where(sel, ref_b, 0.0))

    sel0 = lin == 0
    sx0, sy0, sz0 = [], [], []
    for b in range(B):
        sx0.append(pick(px_ref[b], sel0))
        sy0.append(pick(py_ref[b], sel0))
        sz0.append(pick(pz_ref[b], sel0))
        dx = px_ref[b] - sx0[b]
        dy = py_ref[b] - sy0[b]
        dz = pz_ref[b] - sz0[b]
        d_ref[b] = (dx * dx + dy * dy) + dz * dz
    idx_ref[0:1, :] = jnp.zeros((1, B), jnp.int32)
    sx_ref[0:1, :] = jnp.stack(sx0).reshape(1, B)
    sy_ref[0:1, :] = jnp.stack(sy0).reshape(1, B)
    sz_ref[0:1, :] = jnp.stack(sz0).reshape(1, B)

    def body(i, _):
        nidx, nsx, nsy, nsz = [], [], [], []
        for b in range(B):
            pxb = px_ref[b]
            pyb = py_ref[b]
            pzb = pz_ref[b]
            d = d_ref[b]
            m = jnp.max(d)
            ib = jnp.min(jnp.where(d == m, lin, BIG))
            sel = lin == ib
            sx = pick(pxb, sel)
            sy = pick(pyb, sel)
            sz = pick(pzb, sel)
            dx = pxb - sx
            dy = pyb - sy
            dz = pzb - sz
            nd = (dx * dx + dz * dz) + dy * dy
            d_ref[b] = jnp.minimum(d, nd)
            nidx.append(ib)
            nsx.append(sx)
            nsy.append(sy)
            nsz.append(sz)
        idx_ref[pl.ds(i, 1), :] = jnp.stack(nidx).reshape(1, B)
        sx_ref[pl.ds(i, 1), :] = jnp.stack(nsx).reshape(1, B)
        sy_ref[pl.ds(i, 1), :] = jnp.stack(nsy).reshape(1, B)
        sz_ref[pl.ds(i, 1), :] = jnp.stack(nsz).reshape(1, B)
        return 0

    lax.fori_loop(1, M, body, 0)


def _run_fps(p, M):
    B, N, _ = p.shape
    R = 8
    C = N // R
    px = p[..., 0].reshape(B, R, C)
    py = p[..., 1].reshape(B, R, C)
    pz = p[..., 2].reshape(B, R, C)
    out_shape = [
        jax.ShapeDtypeStruct((M, B), jnp.int32),
        jax.ShapeDtypeStruct((M, B), jnp.float32),
        jax.ShapeDtypeStruct((M, B), jnp.float32),
        jax.ShapeDtypeStruct((M, B), jnp.float32),
    ]
    return pl.pallas_call(
        _fps_body,
        out_shape=out_shape,
        scratch_shapes=[pltpu.VMEM((B, R, C), jnp.float32)],
    )(px, py, pz)


def _mm_body(f_ref, w_ref, y_ref):
    y_ref[...] = jnp.dot(f_ref[...], w_ref[...],
                         preferred_element_type=jnp.float32)


def _run_dense(feats_pad, w_pad):
    BN_, K = feats_pad.shape
    CO = w_pad.shape[1]
    BLK = 2048
    return pl.pallas_call(
        _mm_body,
        grid=(BN_ // BLK,),
        in_specs=[
            pl.BlockSpec((BLK, K), lambda i: (i, 0)),
            pl.BlockSpec((K, CO), lambda i: (0, 0)),
        ],
        out_specs=pl.BlockSpec((BLK, CO), lambda i: (i, 0)),
        out_shape=jax.ShapeDtypeStruct((BN_, CO), jnp.float32),
    )(feats_pad, w_pad)


def _sc_gather_stats(sel2d, knn_flat, y, B, N, M, CO):
    BM = B * M
    CH = BM // _NW
    NCHK = CH * _NS // 128
    PPC = 128 // _NS
    NQ = CO // 16
    wpb = _NW // B

    mesh = plsc.VectorSubcoreMesh(core_axis_name="c", subcore_axis_name="s")

    @functools.partial(
        pl.kernel,
        mesh=mesh,
        compiler_params=pltpu.CompilerParams(use_tc_tiling_on_sc=False),
        out_type=(
            jax.ShapeDtypeStruct((BM, CO), jnp.float32),
            jax.ShapeDtypeStruct((BM, CO), jnp.float32),
            jax.ShapeDtypeStruct((BM, CO), jnp.float32),
            jax.ShapeDtypeStruct((_NW, CO), jnp.float32),
        ),
        scratch_types=[
            pltpu.VMEM((CH // 128, 128), jnp.int32),
            pltpu.VMEM((CH, _NS), jnp.int32),
            pltpu.VMEM((NCHK, 128), jnp.int32),
            pltpu.VMEM((128, CO), jnp.float32),
            pltpu.VMEM((128, CO), jnp.float32),
            pltpu.VMEM((2, PPC, CO), jnp.float32),
            pltpu.VMEM((2, PPC, CO), jnp.float32),
            pltpu.VMEM((2, PPC, CO), jnp.float32),
            pltpu.VMEM((1, CO), jnp.float32),
            pltpu.SemaphoreType.DMA,
            pltpu.SemaphoreType.DMA,
            pltpu.SemaphoreType.DMA,
            pltpu.SemaphoreType.DMA,
            pltpu.SemaphoreType.DMA,
        ],
    )
    def sck(sel_hbm, knn_hbm, y_hbm, gs_hbm, gx_hbm, gn_hbm, g2_hbm,
            sel_v, knn_v, idx_buf, bufa, bufb, wgs, wgx, wgn, g2_v,
            sa, sb, wa, wb, sk):
        wid = lax.axis_index("s") * 2 + lax.axis_index("c")
        base = wid * CH
        boff = (wid // wpb) * N
        nsel = CH // 128

        pltpu.sync_copy(sel_hbm.at[pl.ds(wid * nsel, nsel)], sel_v)
        for ci in range(nsel):
            for q in range(128 // 16):
                sl = (ci, pl.ds(q * 16, 16))
                sel_v[sl] = sel_v[sl] + boff
        knn_cps = []
        for ci in range(nsel):
            knn_cps.append(pltpu.async_copy(
                knn_hbm.at[sel_v.at[ci]],
                knn_v.at[pl.ds(ci * 128, 128)], sk))
        for cp in knn_cps:
            cp.wait()

        def cpbody(c, _):
            for pt in range(PPC):
                row = knn_v[c * PPC + pt, :] + boff
                idx_buf[c, pl.ds(pt * _NS, _NS)] = row
            return 0

        lax.fori_loop(0, NCHK, cpbody, 0)

        pltpu.async_copy(y_hbm.at[idx_buf.at[0]], bufa, sa)
        pltpu.async_copy(y_hbm.at[idx_buf.at[1]], bufb, sb)

        def outer(o, g2c):
            for half in range(2):
                c = o * 2 + half
                buf = bufa if half == 0 else bufb
                sem = sa if half == 0 else sb
                wsem = wa if half == 0 else wb
                pltpu.make_async_copy(y_hbm.at[idx_buf.at[c]], buf,
                                      sem).wait()

                @pl.when(c >= 2)
                def _():
                    for wbuf, hbm in ((wgs, gs_hbm), (wgx, gx_hbm),
                                      (wgn, gn_hbm)):
                        pltpu.make_async_copy(
                            wbuf.at[half], hbm.at[pl.ds(base, PPC)],
                            wsem).wait()

                def ptbody(pt, g2i):
                    r0 = pt * _NS
                    g2o = []
                    for q in range(NQ):
                        qs = pl.ds(q * 16, 16)
                        v = buf[r0, qs]
                        s, mx, mn = v, v, v
                        g2q = g2i[q] + v * v
                        for j in range(1, _NS):
                            v = buf[r0 + j, qs]
                            s = s + v
                            mx = jnp.maximum(mx, v)
                            mn = jnp.minimum(mn, v)
                            g2q = g2q + v * v
                        wgs[half, pt, qs] = s
                        wgx[half, pt, qs] = mx
                        wgn[half, pt, qs] = mn
                        g2o.append(g2q)
                    return tuple(g2o)

                g2c = lax.fori_loop(0, PPC, ptbody, g2c)

                orow0 = base + c * PPC
                for wbuf, hbm in ((wgs, gs_hbm), (wgx, gx_hbm),
                                  (wgn, gn_hbm)):
                    pltpu.async_copy(wbuf.at[half],
                                     hbm.at[pl.ds(orow0, PPC)], wsem)

                @pl.when(c + 2 < NCHK)
                def _fire():
                    pltpu.async_copy(y_hbm.at[idx_buf.at[c + 2]], buf, sem)
            return g2c

        zeros = jnp.zeros((16,), jnp.float32)
        g2 = lax.fori_loop(0, NCHK // 2, outer, (zeros,) * NQ)

        for half in range(2):
            wsem = wa if half == 0 else wb
            for wbuf, hbm in ((wgs, gs_hbm), (wgx, gx_hbm), (wgn, gn_hbm)):
                pltpu.make_async_copy(wbuf.at[half],
                                      hbm.at[pl.ds(base, PPC)], wsem).wait()
        for q in range(NQ):
            g2_v[0, pl.ds(q * 16, 16)] = g2[q]
        pltpu.sync_copy(g2_v, g2_hbm.at[pl.ds(wid, 1)])

    return sck(sel2d, knn_flat, y)


def _fin_body(gs_ref, gx_ref, gn_ref, g2_ref, np8_ref, wx_ref, gam_ref,
              bet_ref, out_ref):
    z = jnp.dot(np8_ref[...], wx_ref[...],
                preferred_element_type=jnp.float32)
    gs = gs_ref[...]
    ns = jnp.float32(_NS)
    s1 = jnp.sum(gs, axis=0) - ns * jnp.sum(z, axis=0)
    s2 = (jnp.sum(g2_ref[...], axis=0)
          - 2.0 * jnp.sum(z * gs, axis=0)
          + ns * jnp.sum(z * z, axis=0))
    cnt = jnp.float32(gs_ref.shape[0] * _NS)
    mean = s1 / cnt
    var = s2 / cnt - mean * mean
    inv = 1.0 / jnp.sqrt(var + _EPS)
    scale = gam_ref[...] * inv
    shift = bet_ref[...] - (mean * scale)
    hsel = jnp.where(scale >= 0.0, gx_ref[...], gn_ref[...]) - z
    out_ref[...] = jnp.maximum(scale * hsel + shift, 0.0)


def _run_finalize(gs, gx, gn, g2, np8, wx8, gamma, beta):
    BM, CO = gs.shape
    return pl.pallas_call(
        _fin_body,
        out_shape=jax.ShapeDtypeStruct((BM, CO), jnp.float32),
        compiler_params=pltpu.CompilerParams(
            vmem_limit_bytes=100 * 1024 * 1024),
    )(gs, gx, gn, g2, np8, wx8, gamma.reshape(1, CO), beta.reshape(1, CO))


def kernel(x, p, knn_ind, W, gamma, beta):
    B, N, CI = x.shape
    M = N // 4
    CO = W.shape[0]
    BM = B * M

    idx_mb, sx_mb, sy_mb, sz_mb = _run_fps(p, M)
    new_p = jnp.stack([sx_mb, sy_mb, sz_mb], axis=-1).transpose(1, 0, 2)

    feats = jnp.concatenate(
        [p, x, jnp.zeros((B, N, 128 - 3 - CI), jnp.float32)], axis=-1)
    w_pad = jnp.zeros((128, CO), jnp.float32).at[:3 + CI].set(W.T)
    y = _run_dense(feats.reshape(B * N, 128), w_pad)

    sel2d = idx_mb.T.reshape(BM // 128, 128)
    knn_flat = knn_ind.reshape(B * N, _NS)
    gs, gx, gn, g2 = _sc_gather_stats(sel2d, knn_flat, y, B, N, M, CO)

    np8 = jnp.zeros((BM, 8), jnp.float32).at[:, :3].set(new_p.reshape(BM, 3))
    wx8 = jnp.zeros((8, CO), jnp.float32).at[:3].set(W[:, :3].T)
    out = _run_finalize(gs, gx, gn, g2, np8, wx8, gamma, beta)

    return (out.reshape(B, M, CO), new_p, knn_ind)


# speedup vs baseline: 2.4913x; 2.4913x over previous
import functools

import jax
import jax.numpy as jnp
from jax import lax
from jax.experimental import pallas as pl
from jax.experimental.pallas import tpu as pltpu
from jax.experimental.pallas import tpu_sc as plsc

_EPS = 1e-5
_NS = 16
_NW = 32


def _fps_body(px_ref, py_ref, pz_ref, idx_ref, sx_ref, sy_ref, sz_ref,
              *d_refs):
    B, R, C = px_ref.shape
    M = idx_ref.shape[0]
    lin = (lax.broadcasted_iota(jnp.int32, (R, C), 0) * C
           + lax.broadcasted_iota(jnp.int32, (R, C), 1))
    BIG = jnp.int32(1 << 30)

    def pick(ref_b, sel):
        return jnp.sum(jnp.where(sel, ref_b, 0.0))

    sel0 = lin == 0
    sx0, sy0, sz0 = [], [], []
    for b in range(B):
        sx0.append(pick(px_ref[b], sel0))
        sy0.append(pick(py_ref[b], sel0))
        sz0.append(pick(pz_ref[b], sel0))
        dx = px_ref[b] - sx0[b]
        dy = py_ref[b] - sy0[b]
        dz = pz_ref[b] - sz0[b]
        d_refs[b][...] = (dx * dx + dy * dy) + dz * dz
    idx_ref[0:1, :] = jnp.zeros((1, B), jnp.int32)
    sx_ref[0:1, :] = jnp.stack(sx0).reshape(1, B)
    sy_ref[0:1, :] = jnp.stack(sy0).reshape(1, B)
    sz_ref[0:1, :] = jnp.stack(sz0).reshape(1, B)

    def body(i, _):
        ds = [d_refs[b][...] for b in range(B)]
        ms = [jnp.max(ds[b]) for b in range(B)]
        ibs = [jnp.min(jnp.where(ds[b] == ms[b], lin, BIG))
               for b in range(B)]
        sels = [lin == ibs[b] for b in range(B)]
        nsx = [pick(px_ref[b], sels[b]) for b in range(B)]
        nsy = [pick(py_ref[b], sels[b]) for b in range(B)]
        nsz = [pick(pz_ref[b], sels[b]) for b in range(B)]
        for b in range(B):
            dx = px_ref[b] - nsx[b]
            dy = py_ref[b] - nsy[b]
            dz = pz_ref[b] - nsz[b]
            nd = (dx * dx + dz * dz) + dy * dy
            d_refs[b][...] = jnp.minimum(ds[b], nd)
        idx_ref[pl.ds(i, 1), :] = jnp.stack(ibs).reshape(1, B)
        sx_ref[pl.ds(i, 1), :] = jnp.stack(nsx).reshape(1, B)
        sy_ref[pl.ds(i, 1), :] = jnp.stack(nsy).reshape(1, B)
        sz_ref[pl.ds(i, 1), :] = jnp.stack(nsz).reshape(1, B)
        return 0

    lax.fori_loop(1, M, body, 0)


def _run_fps(p, M):
    B, N, _ = p.shape
    R = 8
    C = N // R
    px = p[..., 0].reshape(B, R, C)
    py = p[..., 1].reshape(B, R, C)
    pz = p[..., 2].reshape(B, R, C)
    out_shape = [
        jax.ShapeDtypeStruct((M, B), jnp.int32),
        jax.ShapeDtypeStruct((M, B), jnp.float32),
        jax.ShapeDtypeStruct((M, B), jnp.float32),
        jax.ShapeDtypeStruct((M, B), jnp.float32),
    ]
    return pl.pallas_call(
        _fps_body,
        out_shape=out_shape,
        scratch_shapes=[pltpu.VMEM((R, C), jnp.float32) for _ in range(B)],
    )(px, py, pz)


def _mm_body(f_ref, w_ref, y_ref):
    y_ref[...] = jnp.dot(f_ref[...], w_ref[...],
                         preferred_element_type=jnp.float32)


def _run_dense(feats_pad, w_pad):
    BN_, K = feats_pad.shape
    CO = w_pad.shape[1]
    BLK = 2048
    return pl.pallas_call(
        _mm_body,
        grid=(BN_ // BLK,),
        in_specs=[
            pl.BlockSpec((BLK, K), lambda i: (i, 0)),
            pl.BlockSpec((K, CO), lambda i: (0, 0)),
        ],
        out_specs=pl.BlockSpec((BLK, CO), lambda i: (i, 0)),
        out_shape=jax.ShapeDtypeStruct((BN_, CO), jnp.float32),
    )(feats_pad, w_pad)


def _sc_gather_stats(sel2d, knn_flat, y, B, N, M, CO):
    BM = B * M
    CH = BM // _NW
    NCHK = CH * _NS // 128
    PPC = 128 // _NS
    NQ = CO // 16
    wpb = _NW // B

    mesh = plsc.VectorSubcoreMesh(core_axis_name="c", subcore_axis_name="s")

    @functools.partial(
        pl.kernel,
        mesh=mesh,
        compiler_params=pltpu.CompilerParams(use_tc_tiling_on_sc=False),
        out_type=(
            jax.ShapeDtypeStruct((BM, CO), jnp.float32),
            jax.ShapeDtypeStruct((BM, CO), jnp.float32),
            jax.ShapeDtypeStruct((BM, CO), jnp.float32),
            jax.ShapeDtypeStruct((_NW, CO), jnp.float32),
        ),
        scratch_types=[
            pltpu.VMEM((CH // 128, 128), jnp.int32),
            pltpu.VMEM((CH, _NS), jnp.int32),
            pltpu.VMEM((NCHK, 128), jnp.int32),
            pltpu.VMEM((128, CO), jnp.float32),
            pltpu.VMEM((128, CO), jnp.float32),
            pltpu.VMEM((2, PPC, CO), jnp.float32),
            pltpu.VMEM((2, PPC, CO), jnp.float32),
            pltpu.VMEM((2, PPC, CO), jnp.float32),
            pltpu.VMEM((1, CO), jnp.float32),
            pltpu.SemaphoreType.DMA,
            pltpu.SemaphoreType.DMA,
            pltpu.SemaphoreType.DMA,
            pltpu.SemaphoreType.DMA,
            pltpu.SemaphoreType.DMA,
        ],
    )
    def sck(sel_hbm, knn_hbm, y_hbm, gs_hbm, gx_hbm, gn_hbm, g2_hbm,
            sel_v, knn_v, idx_buf, bufa, bufb, wgs, wgx, wgn, g2_v,
            sa, sb, wa, wb, sk):
        wid = lax.axis_index("s") * 2 + lax.axis_index("c")
        base = wid * CH
        boff = (wid // wpb) * N
        nsel = CH // 128

        pltpu.sync_copy(sel_hbm.at[pl.ds(wid * nsel, nsel)], sel_v)
        for ci in range(nsel):
            for q in range(128 // 16):
                sl = (ci, pl.ds(q * 16, 16))
                sel_v[sl] = sel_v[sl] + boff
        knn_cps = []
        for ci in range(nsel):
            knn_cps.append(pltpu.async_copy(
                knn_hbm.at[sel_v.at[ci]],
                knn_v.at[pl.ds(ci * 128, 128)], sk))
        for cp in knn_cps:
            cp.wait()

        def cpbody(c, _):
            for pt in range(PPC):
                row = knn_v[c * PPC + pt, :] + boff
                idx_buf[c, pl.ds(pt * _NS, _NS)] = row
            return 0

        lax.fori_loop(0, NCHK, cpbody, 0)

        pltpu.async_copy(y_hbm.at[idx_buf.at[0]], bufa, sa)
        pltpu.async_copy(y_hbm.at[idx_buf.at[1]], bufb, sb)

        def outer(o, g2c):
            for half in range(2):
                c = o * 2 + half
                buf = bufa if half == 0 else bufb
                sem = sa if half == 0 else sb
                wsem = wa if half == 0 else wb
                pltpu.make_async_copy(y_hbm.at[idx_buf.at[c]], buf,
                                      sem).wait()

                @pl.when(c >= 2)
                def _():
                    for wbuf, hbm in ((wgs, gs_hbm), (wgx, gx_hbm),
                                      (wgn, gn_hbm)):
                        pltpu.make_async_copy(
                            wbuf.at[half], hbm.at[pl.ds(base, PPC)],
                            wsem).wait()

                def ptbody(pt, g2i):
                    r0 = pt * _NS
                    g2o = []
                    for q in range(NQ):
                        qs = pl.ds(q * 16, 16)
                        v = buf[r0, qs]
                        s, mx, mn = v, v, v
                        g2q = g2i[q] + v * v
                        for j in range(1, _NS):
                            v = buf[r0 + j, qs]
                            s = s + v
                            mx = jnp.maximum(mx, v)
                            mn = jnp.minimum(mn, v)
                            g2q = g2q + v * v
                        wgs[half, pt, qs] = s
                        wgx[half, pt, qs] = mx
                        wgn[half, pt, qs] = mn
                        g2o.append(g2q)
                    return tuple(g2o)

                g2c = lax.fori_loop(0, PPC, ptbody, g2c)

                orow0 = base + c * PPC
                for wbuf, hbm in ((wgs, gs_hbm), (wgx, gx_hbm),
                                  (wgn, gn_hbm)):
                    pltpu.async_copy(wbuf.at[half],
                                     hbm.at[pl.ds(orow0, PPC)], wsem)

                @pl.when(c + 2 < NCHK)
                def _fire():
                    pltpu.async_copy(y_hbm.at[idx_buf.at[c + 2]], buf, sem)
            return g2c

        zeros = jnp.zeros((16,), jnp.float32)
        g2 = lax.fori_loop(0, NCHK // 2, outer, (zeros,) * NQ)

        for half in range(2):
            wsem = wa if half == 0 else wb
            for wbuf, hbm in ((wgs, gs_hbm), (wgx, gx_hbm), (wgn, gn_hbm)):
                pltpu.make_async_copy(wbuf.at[half],
                                      hbm.at[pl.ds(base, PPC)], wsem).wait()
        for q in range(NQ):
            g2_v[0, pl.ds(q * 16, 16)] = g2[q]
        pltpu.sync_copy(g2_v, g2_hbm.at[pl.ds(wid, 1)])

    return sck(sel2d, knn_flat, y)


def _fin_body(gs_ref, gx_ref, gn_ref, g2_ref, np8_ref, wx_ref, gam_ref,
              bet_ref, out_ref):
    z = jnp.dot(np8_ref[...], wx_ref[...],
                preferred_element_type=jnp.float32)
    gs = gs_ref[...]
    ns = jnp.float32(_NS)
    s1 = jnp.sum(gs, axis=0) - ns * jnp.sum(z, axis=0)
    s2 = (jnp.sum(g2_ref[...], axis=0)
          - 2.0 * jnp.sum(z * gs, axis=0)
          + ns * jnp.sum(z * z, axis=0))
    cnt = jnp.float32(gs_ref.shape[0] * _NS)
    mean = s1 / cnt
    var = s2 / cnt - mean * mean
    inv = 1.0 / jnp.sqrt(var + _EPS)
    scale = gam_ref[...] * inv
    shift = bet_ref[...] - (mean * scale)
    hsel = jnp.where(scale >= 0.0, gx_ref[...], gn_ref[...]) - z
    out_ref[...] = jnp.maximum(scale * hsel + shift, 0.0)


def _run_finalize(gs, gx, gn, g2, np8, wx8, gamma, beta):
    BM, CO = gs.shape
    return pl.pallas_call(
        _fin_body,
        out_shape=jax.ShapeDtypeStruct((BM, CO), jnp.float32),
        compiler_params=pltpu.CompilerParams(
            vmem_limit_bytes=100 * 1024 * 1024),
    )(gs, gx, gn, g2, np8, wx8, gamma.reshape(1, CO), beta.reshape(1, CO))


def kernel(x, p, knn_ind, W, gamma, beta):
    B, N, CI = x.shape
    M = N // 4
    CO = W.shape[0]
    BM = B * M

    idx_mb, sx_mb, sy_mb, sz_mb = _run_fps(p, M)
    new_p = jnp.stack([sx_mb, sy_mb, sz_mb], axis=-1).transpose(1, 0, 2)

    feats = jnp.concatenate(
        [p, x, jnp.zeros((B, N, 128 - 3 - CI), jnp.float32)], axis=-1)
    w_pad = jnp.zeros((128, CO), jnp.float32).at[:3 + CI].set(W.T)
    y = _run_dense(feats.reshape(B * N, 128), w_pad)

    sel2d = idx_mb.T.reshape(BM // 128, 128)
    knn_flat = knn_ind.reshape(B * N, _NS)
    gs, gx, gn, g2 = _sc_gather_stats(sel2d, knn_flat, y, B, N, M, CO)

    np8 = jnp.zeros((BM, 8), jnp.float32).at[:, :3].set(new_p.reshape(BM, 3))
    wx8 = jnp.zeros((8, CO), jnp.float32).at[:3].set(W[:, :3].T)
    out = _run_finalize(gs, gx, gn, g2, np8, wx8, gamma, beta)

    return (out.reshape(B, M, CO), new_p, knn_ind)
